# baseline (speedup 1.0000x reference)
import functools
import math

import numpy as np
import jax
import jax.numpy as jnp
from jax import lax
from jax.experimental import pallas as pl
from jax.experimental.pallas import tpu as pltpu
from jax.experimental.pallas import tpu_sc as plsc

D = 64
SEQ = 200
CHUNK = 128
SCALE = 8.0


def _pos_embedding(max_len, d_model):
    pe = np.zeros((max_len, d_model), dtype=np.float32)
    position = np.arange(0, max_len, dtype=np.float32)[:, None]
    div_term = np.exp(-np.arange(0, d_model, 2, dtype=np.float32)
                      * (math.log(10000.0) / d_model))
    pe[:, 0::2] = np.sin(position * div_term)
    pe[:, 1::2] = np.cos(position * div_term)
    return pe


@functools.lru_cache(maxsize=None)
def _pe2_const(seq, d):
    pe = _pos_embedding(800, d)[:seq, :]
    return jnp.asarray(np.concatenate([pe, pe], axis=0))


def _make_body(n_rows, rows_per_worker):
    info = plsc.get_sparse_core_info()
    nc, ns = info.num_cores, info.num_subcores
    chunks = rows_per_worker // CHUNK

    mesh = plsc.VectorSubcoreMesh(core_axis_name="c", subcore_axis_name="s")

    @functools.partial(
        pl.kernel,
        mesh=mesh,
        compiler_params=pltpu.CompilerParams(use_tc_tiling_on_sc=False),
        out_type=jax.ShapeDtypeStruct((n_rows, D), jnp.float32),
        scratch_types=[
            pltpu.VMEM((1, CHUNK), jnp.int32),
            pltpu.VMEM((CHUNK, D), jnp.float32),
            pltpu.VMEM((2 * SEQ, D), jnp.float32),
            pltpu.SemaphoreType.DMA,
        ],
    )
    def body(table_hbm, idx_hbm, pe_hbm, out_hbm, idx_v, rows_v, pe_v, sem):
        wid = lax.axis_index("s") * nc + lax.axis_index("c")
        pltpu.sync_copy(pe_hbm, pe_v)

        def step(it, carry):
            base = wid * rows_per_worker + it * CHUNK
            row = base // CHUNK
            pbase = lax.rem(base, SEQ)
            pltpu.sync_copy(idx_hbm.at[pl.ds(row, 1)], idx_v)
            pltpu.async_copy(table_hbm.at[idx_v.at[0]], rows_v, sem).wait()

            def row_step(i, c2):
                pr = pbase + i
                for k in range(D // 16):
                    s = pl.ds(k * 16, 16)
                    rows_v[i, s] = rows_v[i, s] * SCALE + pe_v[pr, s]
                return c2

            lax.fori_loop(0, CHUNK, row_step, 0)
            pltpu.sync_copy(rows_v, out_hbm.at[pl.ds(base, CHUNK)])
            return carry

        lax.fori_loop(0, chunks, step, 0)

    return body


def kernel(x, table):
    b, s = x.shape
    n_rows = b * s
    nw = 32
    rows_per_worker = n_rows // nw
    idx = x.reshape(n_rows // CHUNK, CHUNK)
    pe2 = _pe2_const(s, D)
    body = _make_body(n_rows, rows_per_worker)
    out = body(table, idx, pe2)
    return out.reshape(b, s, D)
